# baseline (speedup 1.0000x reference)
import functools

import jax
import jax.numpy as jnp
from jax import lax
from jax.experimental import pallas as pl
from jax.experimental.pallas import tpu as pltpu
from jax.experimental.pallas import tpu_sc as plsc

B = 4096
L = 200
D = 64
T = B * L

NW = 32
PER_W = T // NW
C = 64
NSLOT = 4
MACRO = PER_W // (C * NSLOT)

_mesh = plsc.VectorSubcoreMesh(core_axis_name="c", subcore_axis_name="s")


@functools.partial(
    pl.kernel,
    mesh=_mesh,
    compiler_params=pltpu.CompilerParams(
        use_tc_tiling_on_sc=False, needs_layout_passes=False
    ),
    out_type=jax.ShapeDtypeStruct((T, D), jnp.float32),
    scratch_types=[
        pltpu.VMEM((NSLOT, C), jnp.int32),
        pltpu.VMEM((NSLOT, C), jnp.int32),
        pltpu.VMEM((NSLOT, C), jnp.int32),
        pltpu.VMEM((NSLOT, C), jnp.int32),
        pltpu.VMEM((NSLOT, C, 2 * D), jnp.float32),
        pltpu.VMEM((NSLOT, C, D), jnp.float32),
        pltpu.VMEM((512, D), jnp.float32),
        pltpu.VMEM((512, D), jnp.float32),
    ]
    + [pltpu.SemaphoreType.DMA] * (3 * NSLOT),
)
def _emb_lookup(tok_hbm, row_hbm, col_hbm, ttab2, rtab, ctab, out_hbm,
                idx_t, idx_p, idx_r, idx_c, buf_p, buf_o, rtab_l, ctab_l,
                *sems):
    s_idx = sems[0:NSLOT]
    s_gat = sems[NSLOT:2 * NSLOT]
    s_out = sems[2 * NSLOT:3 * NSLOT]
    wid = lax.axis_index("s") * 2 + lax.axis_index("c")
    base0 = wid * PER_W

    def issue_idx(j, chunk):
        src = pl.ds(base0 + chunk * C, C)
        pltpu.async_copy(tok_hbm.at[src], idx_t.at[j], s_idx[j])
        pltpu.async_copy(row_hbm.at[src], idx_r.at[j], s_idx[j])
        pltpu.async_copy(col_hbm.at[src], idx_c.at[j], s_idx[j])

    def wait_idx(j):
        pltpu.make_async_copy(tok_hbm.at[pl.ds(0, C)], idx_t.at[j], s_idx[j]).wait()
        pltpu.make_async_copy(row_hbm.at[pl.ds(0, C)], idx_r.at[j], s_idx[j]).wait()
        pltpu.make_async_copy(col_hbm.at[pl.ds(0, C)], idx_c.at[j], s_idx[j]).wait()

    def issue_gather(j):
        for ib in range(C // 16):
            sl = pl.ds(ib * 16, 16)
            idx_p[j, sl] = jax.lax.shift_right_logical(idx_t[j, sl], 1)
        pltpu.async_copy(ttab2.at[idx_p.at[j]], buf_p.at[j], s_gat[j])

    def wait_gather(j):
        pltpu.make_async_copy(ttab2.at[idx_p.at[j]], buf_p.at[j], s_gat[j]).wait()

    def issue_out(j, chunk):
        dst = pl.ds(base0 + chunk * C, C)
        pltpu.async_copy(buf_o.at[j], out_hbm.at[dst], s_out[j])

    def wait_out(j):
        pltpu.make_async_copy(buf_o.at[j], out_hbm.at[pl.ds(0, C)], s_out[j]).wait()

    pltpu.sync_copy(rtab, rtab_l)
    pltpu.sync_copy(ctab, ctab_l)

    for j in range(NSLOT):
        issue_idx(j, j)
    for j in range(NSLOT):
        wait_idx(j)
        issue_gather(j)

    def macro_body(m, carry):
        chunk0 = m * NSLOT
        for j in range(NSLOT):
            wait_gather(j)

            def blk_body(ib, c2):
                i0 = ib * 16
                it16 = idx_t[j, pl.ds(i0, 16)]
                ir16 = idx_r[j, pl.ds(i0, 16)]
                ic16 = idx_c[j, pl.ds(i0, 16)]
                for k in range(16):
                    h64 = (it16[k] & 1) * D
                    ri = ir16[k]
                    ci = ic16[k]
                    for dd in range(D // 16):
                        sl = pl.ds(dd * 16, 16)
                        v = (buf_p[j, i0 + k, pl.ds(h64 + dd * 16, 16)]
                             + rtab_l[ri, sl] + ctab_l[ci, sl])
                        buf_o[j, i0 + k, sl] = v
                return c2

            lax.fori_loop(0, C // 16, blk_body, 0)
            issue_out(j, chunk0 + j)
            @pl.when(m < MACRO - 1)
            def _():
                issue_idx(j, chunk0 + NSLOT + j)

        @pl.when(m < MACRO - 1)
        def _():
            for j in range(NSLOT):
                wait_idx(j)
                wait_out(j)
                issue_gather(j)

        return carry

    lax.fori_loop(0, MACRO, macro_body, 0)
    for j in range(NSLOT):
        wait_out(j)


_TP_BT = 512


def _tp_body(pp_ref, x_ref, o_ref):
    z = lax.dot_general(
        x_ref[...], pp_ref[...], (((1,), (0,)), ((), ())),
        preferred_element_type=jnp.float32)
    h = _TP_BT // 2
    o_ref[:, 0:D] = z[:, 0:h].T
    o_ref[:, D:2 * D] = z[:, h:_TP_BT].T


_N_TP_BLOCKS = -(-1000000 // _TP_BT)

_transpose_tc = pl.pallas_call(
    _tp_body,
    grid=(_N_TP_BLOCKS,),
    in_specs=[
        pl.BlockSpec((_TP_BT, _TP_BT), lambda j: (0, 0)),
        pl.BlockSpec((D, _TP_BT), lambda j: (0, j)),
    ],
    out_specs=pl.BlockSpec((_TP_BT // 2, 2 * D), lambda j: (j, 0)),
    out_shape=jax.ShapeDtypeStruct((500000, 2 * D), jnp.float32),
)


def _pair_select_mats():
    k = lax.broadcasted_iota(jnp.int32, (_TP_BT, _TP_BT), 0)
    c = lax.broadcasted_iota(jnp.int32, (_TP_BT, _TP_BT), 1)
    pp = (c == (k % 2) * (_TP_BT // 2) + k // 2).astype(jnp.float32)
    return pp


def kernel(tokens, row_indices, col_indices, token_table, row_table, col_table):
    tok = tokens.reshape(T).astype(jnp.int32)
    ri = row_indices.reshape(T).astype(jnp.int32)
    ci = col_indices.reshape(T).astype(jnp.int32)
    pp = _pair_select_mats()
    tt2 = _transpose_tc(pp, token_table.T)
    out = _emb_lookup(tok, ri, ci, tt2, row_table, col_table)
    return out.reshape(B, L, D)


# speedup vs baseline: 1.6506x; 1.6506x over previous
import functools

import jax
import jax.numpy as jnp
from jax import lax
from jax.experimental import pallas as pl
from jax.experimental.pallas import tpu as pltpu
from jax.experimental.pallas import tpu_sc as plsc

B = 4096
L = 200
D = 64
T = B * L

NW = 32
PER_W = T // NW
C = 256
NSLOT = 2
MACRO = PER_W // (C * NSLOT)

_mesh = plsc.VectorSubcoreMesh(core_axis_name="c", subcore_axis_name="s")


@functools.partial(
    pl.kernel,
    mesh=_mesh,
    compiler_params=pltpu.CompilerParams(use_tc_tiling_on_sc=False),
    out_type=jax.ShapeDtypeStruct((T, D), jnp.float32),
    scratch_types=[
        pltpu.VMEM((NSLOT, C), jnp.int32),
        pltpu.VMEM((NSLOT, C), jnp.int32),
        pltpu.VMEM((NSLOT, C), jnp.int32),
        pltpu.VMEM((NSLOT, C, D), jnp.float32),
        pltpu.VMEM((NSLOT, C, D), jnp.float32),
        pltpu.VMEM((NSLOT, C, D), jnp.float32),
    ]
    + [pltpu.SemaphoreType.DMA] * (3 * NSLOT),
)
def _emb_lookup(tok_hbm, row_hbm, col_hbm, ttab, rtab, ctab, out_hbm,
                idx_t, idx_r, idx_c, buf_t, buf_r, buf_c, *sems):
    s_idx = sems[0:NSLOT]
    s_gat = sems[NSLOT:2 * NSLOT]
    s_out = sems[2 * NSLOT:3 * NSLOT]
    wid = lax.axis_index("s") * 2 + lax.axis_index("c")
    base0 = wid * PER_W

    def issue_idx(j, chunk):
        src = pl.ds(base0 + chunk * C, C)
        pltpu.async_copy(tok_hbm.at[src], idx_t.at[j], s_idx[j])
        pltpu.async_copy(row_hbm.at[src], idx_r.at[j], s_idx[j])
        pltpu.async_copy(col_hbm.at[src], idx_c.at[j], s_idx[j])

    def wait_idx(j):
        pltpu.make_async_copy(tok_hbm.at[pl.ds(0, C)], idx_t.at[j], s_idx[j]).wait()
        pltpu.make_async_copy(row_hbm.at[pl.ds(0, C)], idx_r.at[j], s_idx[j]).wait()
        pltpu.make_async_copy(col_hbm.at[pl.ds(0, C)], idx_c.at[j], s_idx[j]).wait()

    def issue_gathers(j):
        pltpu.async_copy(ttab.at[idx_t.at[j]], buf_t.at[j], s_gat[j])
        pltpu.async_copy(rtab.at[idx_r.at[j]], buf_r.at[j], s_gat[j])
        pltpu.async_copy(ctab.at[idx_c.at[j]], buf_c.at[j], s_gat[j])

    def wait_gathers(j):
        pltpu.make_async_copy(ttab.at[idx_t.at[j]], buf_t.at[j], s_gat[j]).wait()
        pltpu.make_async_copy(rtab.at[idx_r.at[j]], buf_r.at[j], s_gat[j]).wait()
        pltpu.make_async_copy(ctab.at[idx_c.at[j]], buf_c.at[j], s_gat[j]).wait()

    def issue_out(j, chunk):
        dst = pl.ds(base0 + chunk * C, C)
        pltpu.async_copy(buf_t.at[j], out_hbm.at[dst], s_out[j])

    def wait_out(j):
        pltpu.make_async_copy(buf_t.at[j], out_hbm.at[pl.ds(0, C)], s_out[j]).wait()

    for j in range(NSLOT):
        issue_idx(j, j)
    for j in range(NSLOT):
        wait_idx(j)
        issue_gathers(j)

    def macro_body(m, carry):
        chunk0 = m * NSLOT
        for j in range(NSLOT):
            wait_gathers(j)

            def row_body(i, c2):
                for dd in range(D // 16):
                    sl = pl.ds(dd * 16, 16)
                    v = buf_r[j, i, sl] + buf_c[j, i, sl]
                    plsc.addupdate(buf_t.at[j, i, sl], v)
                return c2

            lax.fori_loop(0, C, row_body, 0, unroll=2)
            issue_out(j, chunk0 + j)
            @pl.when(m < MACRO - 1)
            def _():
                issue_idx(j, chunk0 + NSLOT + j)

        @pl.when(m < MACRO - 1)
        def _():
            for j in range(NSLOT):
                wait_idx(j)
                wait_out(j)
                issue_gathers(j)

        return carry

    lax.fori_loop(0, MACRO, macro_body, 0)
    for j in range(NSLOT):
        wait_out(j)


def kernel(tokens, row_indices, col_indices, token_table, row_table, col_table):
    tok = tokens.reshape(T).astype(jnp.int32)
    ri = row_indices.reshape(T).astype(jnp.int32)
    ci = col_indices.reshape(T).astype(jnp.int32)
    out = _emb_lookup(tok, ri, ci, token_table, row_table, col_table)
    return out.reshape(B, L, D)


# speedup vs baseline: 1.6517x; 1.0007x over previous
import functools

import jax
import jax.numpy as jnp
from jax import lax
from jax.experimental import pallas as pl
from jax.experimental.pallas import tpu as pltpu
from jax.experimental.pallas import tpu_sc as plsc

B = 4096
L = 200
D = 64
T = B * L

NW = 32
PER_W = T // NW
C = 256
NSLOT = 2
MACRO = PER_W // (C * NSLOT)

_mesh = plsc.VectorSubcoreMesh(core_axis_name="c", subcore_axis_name="s")


@functools.partial(
    pl.kernel,
    mesh=_mesh,
    compiler_params=pltpu.CompilerParams(use_tc_tiling_on_sc=False),
    out_type=jax.ShapeDtypeStruct((T, D), jnp.float32),
    scratch_types=[
        pltpu.VMEM((NSLOT, C), jnp.int32),
        pltpu.VMEM((NSLOT, C), jnp.int32),
        pltpu.VMEM((NSLOT, C), jnp.int32),
        pltpu.VMEM((NSLOT, C, D), jnp.float32),
        pltpu.VMEM((NSLOT, C, D), jnp.float32),
        pltpu.VMEM((NSLOT, C, D), jnp.float32),
    ]
    + [pltpu.SemaphoreType.DMA] * (3 * NSLOT),
)
def _emb_lookup(tok_hbm, row_hbm, col_hbm, ttab, rtab, ctab, out_hbm,
                idx_t, idx_r, idx_c, buf_t, buf_r, buf_c, *sems):
    s_idx = sems[0:NSLOT]
    s_gat = sems[NSLOT:2 * NSLOT]
    s_out = sems[2 * NSLOT:3 * NSLOT]
    wid = lax.axis_index("s") * 2 + lax.axis_index("c")
    base0 = wid * PER_W

    def issue_idx(j, chunk):
        src = pl.ds(base0 + chunk * C, C)
        pltpu.async_copy(tok_hbm.at[src], idx_t.at[j], s_idx[j])
        pltpu.async_copy(row_hbm.at[src], idx_r.at[j], s_idx[j])
        pltpu.async_copy(col_hbm.at[src], idx_c.at[j], s_idx[j])

    def wait_idx(j):
        pltpu.make_async_copy(tok_hbm.at[pl.ds(0, C)], idx_t.at[j], s_idx[j]).wait()
        pltpu.make_async_copy(row_hbm.at[pl.ds(0, C)], idx_r.at[j], s_idx[j]).wait()
        pltpu.make_async_copy(col_hbm.at[pl.ds(0, C)], idx_c.at[j], s_idx[j]).wait()

    def issue_gathers(j):
        pltpu.async_copy(ttab.at[idx_t.at[j]], buf_t.at[j], s_gat[j])
        pltpu.async_copy(rtab.at[idx_r.at[j]], buf_r.at[j], s_gat[j])
        pltpu.async_copy(ctab.at[idx_c.at[j]], buf_c.at[j], s_gat[j])

    def wait_gathers(j):
        pltpu.make_async_copy(ttab.at[idx_t.at[j]], buf_t.at[j], s_gat[j]).wait()
        pltpu.make_async_copy(rtab.at[idx_r.at[j]], buf_r.at[j], s_gat[j]).wait()
        pltpu.make_async_copy(ctab.at[idx_c.at[j]], buf_c.at[j], s_gat[j]).wait()

    def issue_out(j, chunk):
        dst = pl.ds(base0 + chunk * C, C)
        pltpu.async_copy(buf_t.at[j], out_hbm.at[dst], s_out[j])

    def wait_out(j):
        pltpu.make_async_copy(buf_t.at[j], out_hbm.at[pl.ds(0, C)], s_out[j]).wait()

    for j in range(NSLOT):
        issue_idx(j, j)
    for j in range(NSLOT):
        wait_idx(j)
        issue_gathers(j)

    def macro_body(m, carry):
        chunk0 = m * NSLOT
        for j in range(NSLOT):
            wait_gathers(j)

            def row_body(i, c2):
                for dd in range(D // 16):
                    sl = pl.ds(dd * 16, 16)
                    v = buf_r[j, i, sl] + buf_c[j, i, sl]
                    plsc.addupdate(buf_t.at[j, i, sl], v)
                return c2

            lax.fori_loop(0, C, row_body, 0, unroll=8)
            issue_out(j, chunk0 + j)
            @pl.when(m < MACRO - 1)
            def _():
                issue_idx(j, chunk0 + NSLOT + j)

        @pl.when(m < MACRO - 1)
        def _():
            for j in range(NSLOT):
                wait_idx(j)
                wait_out(j)
                issue_gathers(j)

        return carry

    lax.fori_loop(0, MACRO, macro_body, 0)
    for j in range(NSLOT):
        wait_out(j)


def kernel(tokens, row_indices, col_indices, token_table, row_table, col_table):
    tok = tokens.reshape(T).astype(jnp.int32)
    ri = row_indices.reshape(T).astype(jnp.int32)
    ci = col_indices.reshape(T).astype(jnp.int32)
    out = _emb_lookup(tok, ri, ci, token_table, row_table, col_table)
    return out.reshape(B, L, D)
